# baseline (speedup 1.0000x reference)
import jax
import jax.numpy as jnp
from jax import lax
from jax.experimental import pallas as pl
from jax.experimental.pallas import tpu as pltpu
from jax.experimental.pallas import tpu_sc as plsc

NUM_FIELDS = 26
VOCAB = 100000
D_MODEL = 32
BATCH = 4096

ESZ = (12416,) * 7 + (VOCAB - 7 * 12416,)
EOF_ = tuple(12416 * k for k in range(8))

NC = 2
NS = 16
NW = NC * NS


def _body(xt_hbm, tab_hbm, bias_hbm, out_hbm, xbuf,
          e0, e1, e2, e3, e4, e5, e6, e7, obuf, biasv,
          s0, s1, s2, s3, s4, s5, s6, s7, sem_x, sem_o):
    ebufs = (e0, e1, e2, e3, e4, e5, e6, e7)
    esems = (s0, s1, s2, s3, s4, s5, s6, s7)
    w = lax.axis_index("s") * NC + lax.axis_index("c")
    pltpu.sync_copy(bias_hbm, biasv)

    def row(f):
        return f * D_MODEL + w

    def ecopy(f, k):
        return pltpu.make_async_copy(
            tab_hbm.at[row(f)].at[pl.ds(EOF_[k], ESZ[k])], ebufs[k], esems[k])

    def start_x(f, p):
        pltpu.make_async_copy(xt_hbm.at[f], xbuf.at[p], sem_x).start()

    for k in range(8):
        ecopy(0, k).start()
    start_x(0, 0)

    def fbody(f, _):
        p = f % 2
        bias_v = plsc.load_gather(
            biasv, [jnp.full((16,), f * D_MODEL, jnp.int32) + w])

        pltpu.make_async_copy(xt_hbm.at[f], xbuf.at[p], sem_x).wait()

        @pl.when(f >= 2)
        def _():
            pltpu.make_async_copy(obuf.at[p], out_hbm.at[row(f)], sem_o).wait()

        def fused_pass(ks, with_bias):
            def one(sl):
                idx = xbuf[p, sl]
                acc = bias_v if with_bias else obuf[p, sl]
                for k in ks:
                    t = idx - EOF_[k]
                    m = t.astype(jnp.uint32) < jnp.uint32(ESZ[k])
                    v = plsc.load_gather(ebufs[k], [jnp.where(m, t, 0)])
                    acc = acc + jnp.where(m, v, 0.0)
                obuf[p, sl] = acc

            def body(i, _):
                for u in range(4):
                    one(pl.ds(i * 64 + u * 16, 16))
                return 0

            lax.fori_loop(0, BATCH // 64, body, 0)

        for k in range(4):
            ecopy(f, k).wait()

        @pl.when(f + 1 < NUM_FIELDS)
        def _():
            start_x(f + 1, 1 - p)

        fused_pass(range(4), True)

        @pl.when(f + 1 < NUM_FIELDS)
        def _():
            for k in range(4):
                ecopy(f + 1, k).start()

        for k in range(4, 8):
            ecopy(f, k).wait()

        fused_pass(range(4, 8), False)

        @pl.when(f + 1 < NUM_FIELDS)
        def _():
            for k in range(4, 8):
                ecopy(f + 1, k).start()

        pltpu.make_async_copy(obuf.at[p], out_hbm.at[row(f)], sem_o).start()
        return 0

    lax.fori_loop(0, NUM_FIELDS, fbody, 0)

    pltpu.make_async_copy(
        obuf.at[0], out_hbm.at[row(NUM_FIELDS - 2)], sem_o).wait()
    pltpu.make_async_copy(
        obuf.at[1], out_hbm.at[row(NUM_FIELDS - 1)], sem_o).wait()


@jax.jit
def _run(xt, tab2d, bias_flat):
    mesh = plsc.VectorSubcoreMesh(core_axis_name="c", subcore_axis_name="s")
    return pl.kernel(
        _body,
        mesh=mesh,
        compiler_params=pltpu.CompilerParams(needs_layout_passes=False),
        out_type=jax.ShapeDtypeStruct((NUM_FIELDS * D_MODEL, BATCH), jnp.float32),
        scratch_types=[
            pltpu.VMEM((2, BATCH), jnp.int32),
            *[pltpu.VMEM((n,), jnp.float32) for n in ESZ],
            pltpu.VMEM((2, BATCH), jnp.float32),
            pltpu.VMEM((NUM_FIELDS * D_MODEL,), jnp.float32),
            *([pltpu.SemaphoreType.DMA] * 8),
            pltpu.SemaphoreType.DMA,
            pltpu.SemaphoreType.DMA,
        ],
    )(xt, tab2d, bias_flat)


def kernel(x, tables, biases):
    xt = x.astype(jnp.int32).T
    tab2d = jnp.transpose(tables, (0, 2, 1)).reshape(
        NUM_FIELDS * D_MODEL, VOCAB)
    out2d = _run(xt, tab2d, biases.reshape(NUM_FIELDS * D_MODEL))
    return out2d.reshape(NUM_FIELDS, D_MODEL, BATCH).transpose(2, 0, 1)


# speedup vs baseline: 1.5821x; 1.5821x over previous
import jax
import jax.numpy as jnp
from jax import lax
from jax.experimental import pallas as pl
from jax.experimental.pallas import tpu as pltpu
from jax.experimental.pallas import tpu_sc as plsc

NUM_FIELDS = 26
VOCAB = 100000
D_MODEL = 32
BATCH = 4096
LO = 49920
HI = VOCAB - LO

NC = 2
NS = 16
NW = NC * NS


def _body(xt_hbm, tab_hbm, bias_hbm, out_hbm, xbuf, tlo, thi, obuf, biasv,
          sem_lo, sem_hi, sem_x, sem_o):
    w = lax.axis_index("s") * NC + lax.axis_index("c")
    pltpu.sync_copy(bias_hbm, biasv)

    def row(f):
        return f * D_MODEL + w

    def start_lo(f, buf):
        pltpu.make_async_copy(
            tab_hbm.at[row(f)].at[pl.ds(0, LO)], buf, sem_lo).start()

    def start_hi(f, buf):
        pltpu.make_async_copy(
            tab_hbm.at[row(f)].at[pl.ds(LO, HI)], buf, sem_hi).start()

    def start_x(f, p):
        pltpu.make_async_copy(xt_hbm.at[f], xbuf.at[p], sem_x).start()

    start_lo(0, tlo)
    start_x(0, 0)
    start_hi(0, thi)

    def fbody(f, _):
        p = f % 2
        bias_v = plsc.load_gather(
            biasv, [jnp.full((16,), f * D_MODEL, jnp.int32) + w])

        pltpu.make_async_copy(xt_hbm.at[f], xbuf.at[p], sem_x).wait()

        @pl.when(f >= 2)
        def _():
            pltpu.make_async_copy(obuf.at[p], out_hbm.at[row(f)], sem_o).wait()

        pltpu.make_async_copy(
            tab_hbm.at[row(f)].at[pl.ds(0, LO)], tlo, sem_lo).wait()

        @pl.when(f + 1 < NUM_FIELDS)
        def _():
            start_x(f + 1, 1 - p)

        def pass_lo(i, _):
            idx = xbuf[p, pl.ds(i * 16, 16)]
            v = plsc.load_gather(tlo, [jnp.minimum(idx, LO - 1)])
            obuf[p, pl.ds(i * 16, 16)] = jnp.where(idx < LO, v, 0.0)
            return 0

        lax.fori_loop(0, BATCH // 16, pass_lo, 0)

        @pl.when(f + 1 < NUM_FIELDS)
        def _():
            start_lo(f + 1, tlo)

        pltpu.make_async_copy(
            tab_hbm.at[row(f)].at[pl.ds(LO, HI)], thi, sem_hi).wait()

        def pass_hi(i, _):
            sl = pl.ds(i * 16, 16)
            idx = xbuf[p, sl]
            ih = jnp.minimum(jnp.maximum(idx - LO, 0), HI - 1)
            v = plsc.load_gather(thi, [ih])
            obuf[p, sl] = obuf[p, sl] + jnp.where(idx >= LO, v, 0.0) + bias_v
            return 0

        lax.fori_loop(0, BATCH // 16, pass_hi, 0)

        @pl.when(f + 1 < NUM_FIELDS)
        def _():
            start_hi(f + 1, thi)

        pltpu.make_async_copy(obuf.at[p], out_hbm.at[row(f)], sem_o).start()
        return 0

    lax.fori_loop(0, NUM_FIELDS, fbody, 0)

    pltpu.make_async_copy(
        obuf.at[0], out_hbm.at[row(NUM_FIELDS - 2)], sem_o).wait()
    pltpu.make_async_copy(
        obuf.at[1], out_hbm.at[row(NUM_FIELDS - 1)], sem_o).wait()


@jax.jit
def _run(xt, tab2d, bias_flat):
    mesh = plsc.VectorSubcoreMesh(core_axis_name="c", subcore_axis_name="s")
    return pl.kernel(
        _body,
        mesh=mesh,
        compiler_params=pltpu.CompilerParams(needs_layout_passes=False),
        out_type=jax.ShapeDtypeStruct((NUM_FIELDS * D_MODEL, BATCH), jnp.float32),
        scratch_types=[
            pltpu.VMEM((2, BATCH), jnp.int32),
            pltpu.VMEM((LO,), jnp.float32),
            pltpu.VMEM((HI,), jnp.float32),
            pltpu.VMEM((2, BATCH), jnp.float32),
            pltpu.VMEM((NUM_FIELDS * D_MODEL,), jnp.float32),
            pltpu.SemaphoreType.DMA,
            pltpu.SemaphoreType.DMA,
            pltpu.SemaphoreType.DMA,
            pltpu.SemaphoreType.DMA,
        ],
    )(xt, tab2d, bias_flat)


def kernel(x, tables, biases):
    xt = x.astype(jnp.int32).T
    tab2d = jnp.transpose(tables, (0, 2, 1)).reshape(
        NUM_FIELDS * D_MODEL, VOCAB)
    out2d = _run(xt, tab2d, biases.reshape(NUM_FIELDS * D_MODEL))
    return out2d.reshape(NUM_FIELDS, D_MODEL, BATCH).transpose(2, 0, 1)
